# baseline (speedup 1.0000x reference)
import functools

import jax
import jax.numpy as jnp
import numpy as np
from jax import lax
from jax.experimental import pallas as pl
from jax.experimental.pallas import tpu as pltpu
from jax.experimental.pallas import tpu_sc as plsc

IN_CH = 16
OUT_CH = 16
HID = 64
POS_DIM = 3
N_POINTS = 10000
N_EDGES = 160000

NC = 2
NS = 16
NW = NC * NS

NCHK = 4
ECHK_RAW = N_EDGES // NCHK
ROWS_PER_W = 10
ECHK = NW * ROWS_PER_W * 128
CHUNK_ROWS = 5
GROUPS = ROWS_PER_W // CHUNK_ROWS
GROW = CHUNK_ROWS * 128

G4 = ECHK // 4

MLP_BLK = 2048

N_ACC = 10016
TRASH = 10000


def _gather_body(tab_sx, tab_pd, src2d, dst2d, gsx, gpd,
                 idx_s, idx_d, buf_sx, buf_pd, sem_s, sem_d):
    c = lax.axis_index("c")
    s = lax.axis_index("s")
    wid = s * NC + c
    row0 = wid * ROWS_PER_W
    ebase = wid * (ROWS_PER_W * 128)
    pltpu.sync_copy(src2d.at[pl.ds(row0, ROWS_PER_W)], idx_s)
    pltpu.sync_copy(dst2d.at[pl.ds(row0, ROWS_PER_W)], idx_d)

    @pl.loop(0, GROUPS)
    def _grp(g):
        cps = []
        for j in range(CHUNK_ROWS):
            r = g * CHUNK_ROWS + j
            cps.append(pltpu.async_copy(
                tab_sx.at[idx_s.at[r]], buf_sx.at[pl.ds(j * 128, 128)], sem_s))
            cps.append(pltpu.async_copy(
                tab_pd.at[idx_d.at[r]], buf_pd.at[pl.ds(j * 128, 128)], sem_d))
        for cp in cps:
            cp.wait()
        off = ebase + g * GROW
        pltpu.sync_copy(buf_sx, gsx.at[pl.ds(off, GROW)])
        pltpu.sync_copy(buf_pd, gpd.at[pl.ds(off, GROW)])


@functools.cache
def _gather():
    return pl.kernel(
        _gather_body,
        out_type=(jax.ShapeDtypeStruct((ECHK, 32), jnp.bfloat16),
                  jax.ShapeDtypeStruct((ECHK, 32), jnp.bfloat16)),
        mesh=plsc.VectorSubcoreMesh(core_axis_name="c", subcore_axis_name="s",
                                    num_cores=NC, num_subcores=NS),
        compiler_params=pltpu.CompilerParams(use_tc_tiling_on_sc=False),
        scratch_types=[
            pltpu.VMEM((ROWS_PER_W, 128), jnp.int32),
            pltpu.VMEM((ROWS_PER_W, 128), jnp.int32),
            pltpu.VMEM((GROW, 32), jnp.bfloat16),
            pltpu.VMEM((GROW, 32), jnp.bfloat16),
            pltpu.SemaphoreType.DMA,
            pltpu.SemaphoreType.DMA,
        ],
    )


SC_ROWS = ECHK // 128
TILE_ROWS = SC_ROWS // NS
SGROUPS = TILE_ROWS // CHUNK_ROWS


def _scatter_body(pair, msg_a, msg_b, dst2d, zeros_hbm, partial,
                  idx, mbuf, acc):
    c = lax.axis_index("c")
    s = lax.axis_index("s")
    zchunk = N_ACC // NS
    pltpu.sync_copy(zeros_hbm.at[pl.ds(s * zchunk, zchunk)],
                    acc.at[pl.ds(s * zchunk, zchunk)])
    plsc.subcore_barrier()
    row0 = s * TILE_ROWS
    ebase = row0 * 128
    drow0 = (2 * pair + c) * (ECHK // 128) + row0
    pltpu.sync_copy(dst2d.at[pl.ds(drow0, TILE_ROWS)], idx)

    def _do(msg16):
        @pl.loop(0, SGROUPS)
        def _grp(g):
            pltpu.sync_copy(msg16.at[pl.ds(ebase + g * GROW, GROW)], mbuf)
            for j in range(CHUNK_ROWS):
                pltpu.sync_copy(mbuf.at[pl.ds(j * 128, 128)],
                                acc.at[idx.at[g * CHUNK_ROWS + j]], add=True)

    @pl.when(c == 0)
    def _():
        _do(msg_a)

    @pl.when(c == 1)
    def _():
        _do(msg_b)

    plsc.subcore_barrier()
    wchunk = N_POINTS // NS
    pltpu.sync_copy(acc.at[pl.ds(s * wchunk, wchunk)],
                    partial.at[c, pl.ds(s * wchunk, wchunk)])


@functools.cache
def _scatter(pair):
    return pl.kernel(
        functools.partial(_scatter_body, pair),
        out_type=jax.ShapeDtypeStruct((NC, N_POINTS, 16), jnp.float32),
        mesh=plsc.VectorSubcoreMesh(core_axis_name="c", subcore_axis_name="s",
                                    num_cores=NC, num_subcores=NS),
        compiler_params=pltpu.CompilerParams(use_tc_tiling_on_sc=False),
        scratch_types=[
            pltpu.VMEM((TILE_ROWS, 128), jnp.int32),
            pltpu.VMEM((GROW, 16), jnp.float32),
            pltpu.VMEM_SHARED((N_ACC, 16), jnp.float32),
        ],
    )


P128 = N_POINTS * 16 // 128


def _combine_body(p_ref, q_ref, o_ref):
    o_ref[...] = (p_ref[0] + p_ref[1]) + (q_ref[0] + q_ref[1])


def _combine(p, q):
    return pl.pallas_call(
        _combine_body,
        in_specs=[pl.BlockSpec((NC, P128, 128), lambda: (0, 0, 0)),
                  pl.BlockSpec((NC, P128, 128), lambda: (0, 0, 0))],
        out_specs=pl.BlockSpec((P128, 128), lambda: (0, 0)),
        out_shape=jax.ShapeDtypeStruct((P128, 128), jnp.float32),
    )(p.reshape(NC, P128, 128), q.reshape(NC, P128, 128))


def _warm_body(z, out, buf):
    c = lax.axis_index("c")
    s = lax.axis_index("s")

    @pl.when(jnp.logical_and(c == 0, s == 0))
    def _():
        pltpu.sync_copy(z.at[pl.ds(0, 16)], buf)
        pltpu.sync_copy(buf, out)


@functools.cache
def _warm():
    return pl.kernel(
        _warm_body,
        out_type=jax.ShapeDtypeStruct((16, 16), jnp.float32),
        mesh=plsc.VectorSubcoreMesh(core_axis_name="c", subcore_axis_name="s",
                                    num_cores=NC, num_subcores=NS),
        compiler_params=pltpu.CompilerParams(use_tc_tiling_on_sc=False,
                                             has_side_effects=True),
        scratch_types=[
            pltpu.VMEM((16, 16), jnp.float32),
        ],
    )


_INV_SQRT2 = np.float32(1.0 / np.sqrt(2.0))


def _mlp_body(gsx_ref, gpd_ref, w1a_ref, w1b_ref, b1_ref, wout_ref, bout_ref,
              t128_ref, r_ref, msg_ref):
    gsb = gsx_ref[...]
    gpb = gpd_ref[...]
    h = jnp.dot(gsb, w1a_ref[...], preferred_element_type=jnp.float32)
    h += jnp.dot(gpb, w1b_ref[...], preferred_element_type=jnp.float32)
    h += b1_ref[...]
    h = 0.5 * h * (1.0 + jax.lax.erf(h * _INV_SQRT2))
    t = jnp.dot(h.astype(jnp.bfloat16), wout_ref[...],
                preferred_element_type=jnp.float32)
    t += bout_ref[...]
    xx = jnp.dot(gsb, t128_ref[...], preferred_element_type=jnp.float32)
    msg_ref[...] = jnp.dot((t * xx).astype(jnp.bfloat16), r_ref[...],
                           preferred_element_type=jnp.float32)


def _edge_mlp(gsx, gpd, w1a, w1b, b1t, wout, boutt, t128, rbd):
    grid = (G4 // MLP_BLK,)
    full = lambda shape: pl.BlockSpec(shape, lambda i: (0, 0))
    return pl.pallas_call(
        _mlp_body,
        grid=grid,
        in_specs=[
            pl.BlockSpec((MLP_BLK, 128), lambda i: (i, 0)),
            pl.BlockSpec((MLP_BLK, 128), lambda i: (i, 0)),
            full((128, 4 * HID)),
            full((128, 4 * HID)),
            full((1, 4 * HID)),
            full((4 * HID, 1024)),
            full((1, 1024)),
            full((128, 1024)),
            full((1024, 64)),
        ],
        out_specs=pl.BlockSpec((MLP_BLK, 64), lambda i: (i, 0)),
        out_shape=jax.ShapeDtypeStruct((G4, 64), jnp.float32),
    )(gsx, gpd, w1a, w1b, b1t, wout, boutt, t128, rbd)


_T128_np = np.zeros((128, 1024), np.float32)
for _k in range(4):
    for _i in range(16):
        _T128_np[32 * _k + 16 + _i,
                 256 * _k + 16 * _i:256 * _k + 16 * _i + 16] = 1.0
_R_np = np.zeros((256, 16), np.float32)
for _i in range(16):
    _R_np[16 * _i:16 * _i + 16, :] += np.eye(16, dtype=np.float32)
_RBD_np = np.kron(np.eye(4, dtype=np.float32), _R_np)


def _bd4(w):
    return jnp.kron(jnp.eye(4, dtype=jnp.float32), w)


def kernel(x, pos, edge_index, W1, b1, W_out, b_out):
    src = edge_index[0].astype(jnp.int32)
    dst = edge_index[1].astype(jnp.int32)
    xf = x.reshape(N_POINTS, IN_CH)
    pos_pad = jnp.pad(pos, ((0, 0), (0, 16 - POS_DIM)))
    tab_sx = jnp.concatenate([pos_pad, xf], axis=1).astype(jnp.bfloat16)
    tab_pd = jnp.pad(pos, ((0, 0), (0, 32 - POS_DIM))).astype(jnp.bfloat16)
    w1a32 = jnp.zeros((32, HID), jnp.float32).at[0:POS_DIM].set(W1[0:POS_DIM])
    w1b32 = jnp.zeros((32, HID), jnp.float32).at[0:POS_DIM].set(W1[POS_DIM:2 * POS_DIM])
    w1a = _bd4(w1a32).astype(jnp.bfloat16)
    w1b = _bd4(w1b32).astype(jnp.bfloat16)
    wout = _bd4(W_out).astype(jnp.bfloat16)
    b1t = jnp.tile(b1, 4)[None, :]
    boutt = jnp.tile(b_out, 4)[None, :]
    t128 = jnp.asarray(_T128_np).astype(jnp.bfloat16)
    rbd = jnp.asarray(_RBD_np).astype(jnp.bfloat16)

    zeros = jnp.zeros((N_ACC, 16), jnp.float32)
    _warm()(zeros)

    pad_n = NCHK * ECHK - N_EDGES
    src_p = jnp.pad(src, (0, pad_n)).reshape(NCHK * SC_ROWS, 128)
    dst_g = jnp.pad(dst, (0, pad_n)).reshape(NCHK * SC_ROWS, 128)
    dst_s = jnp.pad(dst, (0, pad_n),
                    constant_values=TRASH).reshape(NCHK * SC_ROWS, 128)

    msgs = []
    for k in range(NCHK):
        rs = slice(k * SC_ROWS, (k + 1) * SC_ROWS)
        gsx, gpd = _gather()(tab_sx, tab_pd, src_p[rs], dst_g[rs])
        msg64 = _edge_mlp(gsx.reshape(G4, 128), gpd.reshape(G4, 128),
                          w1a, w1b, b1t, wout, boutt, t128, rbd)
        msgs.append(msg64.reshape(ECHK, 16))

    p1 = _scatter(0)(msgs[0], msgs[1], dst_s, zeros)
    p2 = _scatter(1)(msgs[2], msgs[3], dst_s, zeros)
    out = _combine(p1, p2)
    return out.reshape(1, N_POINTS, OUT_CH)


# speedup vs baseline: 1.2137x; 1.2137x over previous
import functools

import jax
import jax.numpy as jnp
import numpy as np
from jax import lax
from jax.experimental import pallas as pl
from jax.experimental.pallas import tpu as pltpu
from jax.experimental.pallas import tpu_sc as plsc

IN_CH = 16
OUT_CH = 16
HID = 64
POS_DIM = 3
N_POINTS = 10000
N_EDGES = 160000

NC = 2
NS = 16
NW = NC * NS

NCHK = 4
ECHK_RAW = N_EDGES // NCHK
ROWS_PER_W = 10
ECHK = NW * ROWS_PER_W * 128
CHUNK_ROWS = 5
GROUPS = ROWS_PER_W // CHUNK_ROWS
GROW = CHUNK_ROWS * 128

G4 = ECHK // 4

MLP_BLK = 2048

N_ACC = 10016
TRASH = 10000


def _gather_body(tab_sx, tab_pd, src2d, dst2d, gsx, gpd,
                 idx_s, idx_d, buf_sx, buf_pd, sem_s, sem_d):
    c = lax.axis_index("c")
    s = lax.axis_index("s")
    wid = s * NC + c
    row0 = wid * ROWS_PER_W
    ebase = wid * (ROWS_PER_W * 128)
    pltpu.sync_copy(src2d.at[pl.ds(row0, ROWS_PER_W)], idx_s)
    pltpu.sync_copy(dst2d.at[pl.ds(row0, ROWS_PER_W)], idx_d)

    @pl.loop(0, GROUPS)
    def _grp(g):
        cps = []
        for j in range(CHUNK_ROWS):
            r = g * CHUNK_ROWS + j
            cps.append(pltpu.async_copy(
                tab_sx.at[idx_s.at[r]], buf_sx.at[pl.ds(j * 128, 128)], sem_s))
            cps.append(pltpu.async_copy(
                tab_pd.at[idx_d.at[r]], buf_pd.at[pl.ds(j * 128, 128)], sem_d))
        for cp in cps:
            cp.wait()
        off = ebase + g * GROW
        pltpu.sync_copy(buf_sx, gsx.at[pl.ds(off, GROW)])
        pltpu.sync_copy(buf_pd, gpd.at[pl.ds(off, GROW)])


@functools.cache
def _gather():
    return pl.kernel(
        _gather_body,
        out_type=(jax.ShapeDtypeStruct((ECHK, 32), jnp.float32),
                  jax.ShapeDtypeStruct((ECHK, 32), jnp.float32)),
        mesh=plsc.VectorSubcoreMesh(core_axis_name="c", subcore_axis_name="s",
                                    num_cores=NC, num_subcores=NS),
        compiler_params=pltpu.CompilerParams(use_tc_tiling_on_sc=False),
        scratch_types=[
            pltpu.VMEM((ROWS_PER_W, 128), jnp.int32),
            pltpu.VMEM((ROWS_PER_W, 128), jnp.int32),
            pltpu.VMEM((GROW, 32), jnp.float32),
            pltpu.VMEM((GROW, 32), jnp.float32),
            pltpu.SemaphoreType.DMA,
            pltpu.SemaphoreType.DMA,
        ],
    )


SC_ROWS = ECHK // 128
TILE_ROWS = SC_ROWS // NS
SGROUPS = TILE_ROWS // CHUNK_ROWS


def _scatter_body(pair, msg_a, msg_b, dst2d, zeros_hbm, partial,
                  idx, mbuf, acc):
    c = lax.axis_index("c")
    s = lax.axis_index("s")
    zchunk = N_ACC // NS
    pltpu.sync_copy(zeros_hbm.at[pl.ds(s * zchunk, zchunk)],
                    acc.at[pl.ds(s * zchunk, zchunk)])
    plsc.subcore_barrier()
    row0 = s * TILE_ROWS
    ebase = row0 * 128
    drow0 = (2 * pair + c) * (ECHK // 128) + row0
    pltpu.sync_copy(dst2d.at[pl.ds(drow0, TILE_ROWS)], idx)

    def _do(msg16):
        @pl.loop(0, SGROUPS)
        def _grp(g):
            pltpu.sync_copy(msg16.at[pl.ds(ebase + g * GROW, GROW)], mbuf)
            for j in range(CHUNK_ROWS):
                pltpu.sync_copy(mbuf.at[pl.ds(j * 128, 128)],
                                acc.at[idx.at[g * CHUNK_ROWS + j]], add=True)

    @pl.when(c == 0)
    def _():
        _do(msg_a)

    @pl.when(c == 1)
    def _():
        _do(msg_b)

    plsc.subcore_barrier()
    wchunk = N_POINTS // NS
    pltpu.sync_copy(acc.at[pl.ds(s * wchunk, wchunk)],
                    partial.at[c, pl.ds(s * wchunk, wchunk)])


@functools.cache
def _scatter(pair):
    return pl.kernel(
        functools.partial(_scatter_body, pair),
        out_type=jax.ShapeDtypeStruct((NC, N_POINTS, 16), jnp.float32),
        mesh=plsc.VectorSubcoreMesh(core_axis_name="c", subcore_axis_name="s",
                                    num_cores=NC, num_subcores=NS),
        compiler_params=pltpu.CompilerParams(use_tc_tiling_on_sc=False),
        scratch_types=[
            pltpu.VMEM((TILE_ROWS, 128), jnp.int32),
            pltpu.VMEM((GROW, 16), jnp.float32),
            pltpu.VMEM_SHARED((N_ACC, 16), jnp.float32),
        ],
    )


P128 = N_POINTS * 16 // 128


def _combine_body(p_ref, q_ref, o_ref):
    o_ref[...] = (p_ref[0] + p_ref[1]) + (q_ref[0] + q_ref[1])


def _combine(p, q):
    return pl.pallas_call(
        _combine_body,
        in_specs=[pl.BlockSpec((NC, P128, 128), lambda: (0, 0, 0)),
                  pl.BlockSpec((NC, P128, 128), lambda: (0, 0, 0))],
        out_specs=pl.BlockSpec((P128, 128), lambda: (0, 0)),
        out_shape=jax.ShapeDtypeStruct((P128, 128), jnp.float32),
    )(p.reshape(NC, P128, 128), q.reshape(NC, P128, 128))


def _warm_body(z, out, buf):
    c = lax.axis_index("c")
    s = lax.axis_index("s")

    @pl.when(jnp.logical_and(c == 0, s == 0))
    def _():
        pltpu.sync_copy(z.at[pl.ds(0, 16)], buf)
        pltpu.sync_copy(buf, out)


@functools.cache
def _warm():
    return pl.kernel(
        _warm_body,
        out_type=jax.ShapeDtypeStruct((16, 16), jnp.float32),
        mesh=plsc.VectorSubcoreMesh(core_axis_name="c", subcore_axis_name="s",
                                    num_cores=NC, num_subcores=NS),
        compiler_params=pltpu.CompilerParams(use_tc_tiling_on_sc=False,
                                             has_side_effects=True),
        scratch_types=[
            pltpu.VMEM((16, 16), jnp.float32),
        ],
    )


_INV_SQRT2 = np.float32(1.0 / np.sqrt(2.0))


def _mlp_body(gsx_ref, gpd_ref, w1a_ref, w1b_ref, b1_ref, wout_ref, bout_ref,
              t128_ref, r_ref, msg_ref):
    gsb = gsx_ref[...].astype(jnp.bfloat16)
    gpb = gpd_ref[...].astype(jnp.bfloat16)
    h = jnp.dot(gsb, w1a_ref[...], preferred_element_type=jnp.float32)
    h += jnp.dot(gpb, w1b_ref[...], preferred_element_type=jnp.float32)
    h += b1_ref[...]
    h = 0.5 * h * (1.0 + jax.lax.erf(h * _INV_SQRT2))
    t = jnp.dot(h.astype(jnp.bfloat16), wout_ref[...],
                preferred_element_type=jnp.float32)
    t += bout_ref[...]
    xx = jnp.dot(gsb, t128_ref[...], preferred_element_type=jnp.float32)
    msg_ref[...] = jnp.dot((t * xx).astype(jnp.bfloat16), r_ref[...],
                           preferred_element_type=jnp.float32)


def _edge_mlp(gsx, gpd, w1a, w1b, b1t, wout, boutt, t128, rbd):
    grid = (G4 // MLP_BLK,)
    full = lambda shape: pl.BlockSpec(shape, lambda i: (0, 0))
    return pl.pallas_call(
        _mlp_body,
        grid=grid,
        in_specs=[
            pl.BlockSpec((MLP_BLK, 128), lambda i: (i, 0)),
            pl.BlockSpec((MLP_BLK, 128), lambda i: (i, 0)),
            full((128, 4 * HID)),
            full((128, 4 * HID)),
            full((1, 4 * HID)),
            full((4 * HID, 1024)),
            full((1, 1024)),
            full((128, 1024)),
            full((1024, 64)),
        ],
        out_specs=pl.BlockSpec((MLP_BLK, 64), lambda i: (i, 0)),
        out_shape=jax.ShapeDtypeStruct((G4, 64), jnp.float32),
    )(gsx, gpd, w1a, w1b, b1t, wout, boutt, t128, rbd)


_T128_np = np.zeros((128, 1024), np.float32)
for _k in range(4):
    for _i in range(16):
        _T128_np[32 * _k + 16 + _i,
                 256 * _k + 16 * _i:256 * _k + 16 * _i + 16] = 1.0
_R_np = np.zeros((256, 16), np.float32)
for _i in range(16):
    _R_np[16 * _i:16 * _i + 16, :] += np.eye(16, dtype=np.float32)
_RBD_np = np.kron(np.eye(4, dtype=np.float32), _R_np)


def _bd4(w):
    return jnp.kron(jnp.eye(4, dtype=jnp.float32), w)


def kernel(x, pos, edge_index, W1, b1, W_out, b_out):
    src = edge_index[0].astype(jnp.int32)
    dst = edge_index[1].astype(jnp.int32)
    xf = x.reshape(N_POINTS, IN_CH)
    pos_pad = jnp.pad(pos, ((0, 0), (0, 16 - POS_DIM)))
    tab_sx = jnp.concatenate([pos_pad, xf], axis=1)
    tab_pd = jnp.pad(pos, ((0, 0), (0, 32 - POS_DIM)))
    w1a32 = jnp.zeros((32, HID), jnp.float32).at[0:POS_DIM].set(W1[0:POS_DIM])
    w1b32 = jnp.zeros((32, HID), jnp.float32).at[0:POS_DIM].set(W1[POS_DIM:2 * POS_DIM])
    w1a = _bd4(w1a32).astype(jnp.bfloat16)
    w1b = _bd4(w1b32).astype(jnp.bfloat16)
    wout = _bd4(W_out).astype(jnp.bfloat16)
    b1t = jnp.tile(b1, 4)[None, :]
    boutt = jnp.tile(b_out, 4)[None, :]
    t128 = jnp.asarray(_T128_np).astype(jnp.bfloat16)
    rbd = jnp.asarray(_RBD_np).astype(jnp.bfloat16)

    zeros = jnp.zeros((N_ACC, 16), jnp.float32)
    _warm()(zeros)

    pad_n = NCHK * ECHK - N_EDGES
    src_p = jnp.pad(src, (0, pad_n)).reshape(NCHK * SC_ROWS, 128)
    dst_g = jnp.pad(dst, (0, pad_n)).reshape(NCHK * SC_ROWS, 128)
    dst_s = jnp.pad(dst, (0, pad_n),
                    constant_values=TRASH).reshape(NCHK * SC_ROWS, 128)

    msgs = []
    for k in range(NCHK):
        rs = slice(k * SC_ROWS, (k + 1) * SC_ROWS)
        gsx, gpd = _gather()(tab_sx, tab_pd, src_p[rs], dst_g[rs])
        msg64 = _edge_mlp(gsx.reshape(G4, 128), gpd.reshape(G4, 128),
                          w1a, w1b, b1t, wout, boutt, t128, rbd)
        msgs.append(msg64.reshape(ECHK, 16))

    p1 = _scatter(0)(msgs[0], msgs[1], dst_s, zeros)
    p2 = _scatter(1)(msgs[2], msgs[3], dst_s, zeros)
    out = _combine(p1, p2)
    return out.reshape(1, N_POINTS, OUT_CH)
